# baseline (speedup 1.0000x reference)
import jax
import jax.numpy as jnp
from jax.experimental import pallas as pl
from jax.experimental.pallas import tpu as pltpu

_N_OUT = 4096


def _mm_body(x_ref, w_ref, b_ref, o_ref):
    xb = x_ref[...].astype(jnp.bfloat16)
    o_ref[...] = (
        jnp.dot(xb, w_ref[...], preferred_element_type=jnp.float32)
        + b_ref[...]
    )


def _pick_tile(total, cap, align):
    best = align
    t = align
    while t <= min(total, cap):
        if total % t == 0:
            best = t
        t += align
    return best


def kernel(x, w_t, b2):
    B, K = x.shape
    Kp, Np = w_t.shape
    assert Kp == K

    wb = w_t.astype(jnp.bfloat16)

    bm = _pick_tile(B, 256, 8)
    bn = _pick_tile(Np, 2048, 128)
    grid = (Np // bn, B // bm)

    out = pl.pallas_call(
        _mm_body,
        grid=grid,
        in_specs=[
            pl.BlockSpec((bm, K), lambda j, i: (i, 0)),
            pl.BlockSpec((K, bn), lambda j, i: (0, j)),
            pl.BlockSpec((1, bn), lambda j, i: (0, j)),
        ],
        out_specs=pl.BlockSpec((bm, bn), lambda j, i: (i, j)),
        out_shape=jax.ShapeDtypeStruct((B, Np), jnp.float32),
        compiler_params=pltpu.CompilerParams(
            dimension_semantics=("parallel", "arbitrary"),
            vmem_limit_bytes=60000 * 1024,
        ),
        cost_estimate=pl.CostEstimate(
            flops=2 * B * Np * K,
            transcendentals=0,
            bytes_accessed=2 * (B * K * (Np // bn) + K * Np) + 4 * B * Np,
        ),
    )(x, wb, b2)

    if Np != _N_OUT:
        out = out[:, :_N_OUT]
    return out


# speedup vs baseline: 1.0114x; 1.0114x over previous
import functools

import jax
import jax.numpy as jnp
from jax.experimental import pallas as pl
from jax.experimental.pallas import tpu as pltpu

_N_OUT = 4096


def _mm_body(x_ref, w_hbm, b_ref, o_ref, w_vmem, sem, *, bn):
    j = pl.program_id(0)
    i = pl.program_id(1)

    @pl.when(i == 0)
    def _():
        cp = pltpu.make_async_copy(
            w_hbm.at[:, pl.ds(j * bn, bn)], w_vmem, sem)
        cp.start()
        cp.wait()

    xb = x_ref[...].astype(jnp.bfloat16)
    o_ref[...] = (
        jnp.dot(xb, w_vmem[...], preferred_element_type=jnp.float32)
        + b_ref[...]
    )


def _pick_tile(total, cap, align):
    best = align
    t = align
    while t <= min(total, cap):
        if total % t == 0:
            best = t
        t += align
    return best


def kernel(x, w_t, b2):
    B, K = x.shape
    Kp, Np = w_t.shape
    assert Kp == K

    wb = w_t.astype(jnp.bfloat16)

    bm = _pick_tile(B, 512, 8)
    bn = _pick_tile(Np, 2048, 128)
    grid = (Np // bn, B // bm)

    out = pl.pallas_call(
        functools.partial(_mm_body, bn=bn),
        grid=grid,
        in_specs=[
            pl.BlockSpec((bm, K), lambda j, i: (i, 0)),
            pl.BlockSpec(memory_space=pl.ANY),
            pl.BlockSpec((1, bn), lambda j, i: (0, j)),
        ],
        out_specs=pl.BlockSpec((bm, bn), lambda j, i: (i, j)),
        out_shape=jax.ShapeDtypeStruct((B, Np), jnp.float32),
        scratch_shapes=[
            pltpu.VMEM((K, bn), jnp.bfloat16),
            pltpu.SemaphoreType.DMA,
        ],
        compiler_params=pltpu.CompilerParams(
            dimension_semantics=("parallel", "arbitrary"),
            vmem_limit_bytes=60000 * 1024,
        ),
        cost_estimate=pl.CostEstimate(
            flops=2 * B * Np * K,
            transcendentals=0,
            bytes_accessed=4 * B * K + 2 * K * Np + 4 * B * Np,
        ),
    )(x, wb, b2)

    if Np != _N_OUT:
        out = out[:, :_N_OUT]
    return out
